# baseline (speedup 1.0000x reference)
import functools

import jax
import jax.numpy as jnp
import numpy as np
from jax import lax
from jax.experimental import pallas as pl
from jax.experimental.shard_map import shard_map
from jax.sharding import Mesh, PartitionSpec as P

N = 4096
BM = 256


def _k1_body(x_ref, w1_ref, b1_ref, w2_ref, adj_ref, out_ref):
    a1 = jnp.dot(adj_ref[...], x_ref[...], preferred_element_type=jnp.float32)
    h1 = jnp.tanh(jnp.dot(a1, w1_ref[...],
                          preferred_element_type=jnp.float32) + b1_ref[...])
    t2 = jnp.dot(h1, w2_ref[...], preferred_element_type=jnp.float32)
    out_ref[...] = t2.astype(jnp.bfloat16)


def _k2_body(t2_ref, b2_ref, w3_ref, adj_ref, out_ref):
    a2 = jnp.dot(adj_ref[...], t2_ref[...], preferred_element_type=jnp.float32)
    h2 = jnp.tanh(a2 + b2_ref[...])
    t3 = jnp.dot(h2, w3_ref[...], preferred_element_type=jnp.float32)
    out_ref[...] = t3.astype(jnp.bfloat16)


def _k3_body(t3_ref, b3_ref, f1w_ref, f1b_ref, f2w_ref, f2b_ref, f3w_ref,
             f3b_ref, adj_ref, out_ref):
    h3 = jnp.dot(adj_ref[...], t3_ref[...],
                 preferred_element_type=jnp.float32) + b3_ref[...]
    a = jnp.maximum(
        jnp.dot(h3, f1w_ref[...], preferred_element_type=jnp.float32)
        + f1b_ref[...], 0.0)
    a = jnp.maximum(
        jnp.dot(a, f2w_ref[...], preferred_element_type=jnp.float32)
        + f2b_ref[...], 0.0)
    out_ref[...] = (jnp.dot(a, f3w_ref[...],
                            preferred_element_type=jnp.float32) + f3b_ref[...])


def _full(shape):
    return pl.BlockSpec(shape, lambda i: (0,) * len(shape))


def _rows(width):
    return pl.BlockSpec((BM, width), lambda i: (i, 0))


def _local_net(x_bf, adj_bf, w1, b1, w2, b2, w3, b3,
               f1w, f1b, f2w, f2b, f3w, f3b):
    m = adj_bf.shape[0]
    grid = (m // BM,)
    bf = jnp.bfloat16

    t2_loc = pl.pallas_call(
        _k1_body, grid=grid,
        in_specs=[_full((N, 512)), _full((512, 1024)), _full((1, 1024)),
                  _full((1024, 512)), _rows(N)],
        out_specs=_rows(512),
        out_shape=jax.ShapeDtypeStruct((m, 512), bf),
    )(x_bf, w1, b1, w2, adj_bf)
    t2 = lax.all_gather(t2_loc, 'r', axis=0, tiled=True)

    t3_loc = pl.pallas_call(
        _k2_body, grid=grid,
        in_specs=[_full((N, 512)), _full((1, 512)), _full((512, 128)),
                  _rows(N)],
        out_specs=_rows(128),
        out_shape=jax.ShapeDtypeStruct((m, 128), bf),
    )(t2, b2, w3, adj_bf)
    t3 = lax.all_gather(t3_loc, 'r', axis=0, tiled=True)

    return pl.pallas_call(
        _k3_body, grid=grid,
        in_specs=[_full((N, 128)), _full((1, 128)),
                  _full((128, 256)), _full((1, 256)),
                  _full((256, 128)), _full((1, 128)),
                  _full((128, 128)), _full((1, 128)), _rows(N)],
        out_specs=_rows(128),
        out_shape=jax.ShapeDtypeStruct((m, 128), jnp.float32),
    )(t3, b3, f1w, f1b, f2w, f2b, f3w, f3b, adj_bf)


def kernel(x, adj, W1, b1, W2, b2, W3, b3,
           fc1_w, fc1_b, fc2_w, fc2_b, fc3_w, fc3_b):
    bf = jnp.bfloat16
    adj_bf = adj.astype(bf)
    x_bf = x.astype(bf)

    f1w = jnp.zeros((128, 256), jnp.float32).at[:, :152].set(fc1_w.T)
    f1b = jnp.zeros((1, 256), jnp.float32).at[0, :152].set(fc1_b)
    f2w = jnp.zeros((256, 128), jnp.float32).at[:152, :48].set(fc2_w.T)
    f2b = jnp.zeros((1, 128), jnp.float32).at[0, :48].set(fc2_b)
    f3w = jnp.zeros((128, 128), jnp.float32).at[:48, :1].set(fc3_w.T)
    f3b = jnp.zeros((1, 128), jnp.float32).at[0, :1].set(fc3_b)

    devs = jax.devices()
    ndev = 2 if len(devs) >= 2 else 1
    mesh = Mesh(np.array(devs[:ndev]), ('r',))
    rep = (P(),) * 12
    net = shard_map(
        _local_net, mesh=mesh,
        in_specs=(P(), P('r', None)) + rep,
        out_specs=P('r', None),
        check_rep=False,
    )
    out = net(x_bf, adj_bf, W1, b1.reshape(1, -1), W2, b2.reshape(1, -1),
              W3, b3.reshape(1, -1), f1w, f1b, f2w, f2b, f3w, f3b)
    return out[:, :1]


# speedup vs baseline: 5.1701x; 5.1701x over previous
import jax
import jax.numpy as jnp
from jax import lax
from jax.experimental import pallas as pl
from jax.experimental.pallas import tpu as pltpu

N = 4096
BM = 256
NB = N // BM


def _body(x_ref, adj_ref, w1_ref, b1_ref, w2_ref, b2_ref, w3_ref, b3_ref,
          f1w_ref, f1b_ref, f2w_ref, f2b_ref, f3w_ref, f3b_ref,
          out_ref, adj_bf, x_bf, u2, u3):
    p = pl.program_id(0)
    i = pl.program_id(1)
    rows = pl.ds(i * BM, BM)
    bf = jnp.bfloat16

    @pl.when(p == 0)
    def _load():
        adj_bf[rows, :] = adj_ref[...].astype(bf)
        x_bf[rows, :] = x_ref[...].astype(bf)

    @pl.when(p == 1)
    def _layer1():
        a1 = jnp.dot(adj_bf[rows, :], x_bf[...],
                     preferred_element_type=jnp.float32)
        h1 = jnp.tanh(jnp.dot(a1, w1_ref[...],
                              preferred_element_type=jnp.float32) + b1_ref[...])
        u2[rows, :] = jnp.dot(h1, w2_ref[...],
                              preferred_element_type=jnp.float32).astype(bf)

    @pl.when(p == 2)
    def _layer2():
        a2 = jnp.dot(adj_bf[rows, :], u2[...],
                     preferred_element_type=jnp.float32)
        h2 = jnp.tanh(a2 + b2_ref[...])
        u3[rows, :] = jnp.dot(h2, w3_ref[...],
                              preferred_element_type=jnp.float32).astype(bf)

    @pl.when(p == 3)
    def _layer3_head():
        h3 = jnp.dot(adj_bf[rows, :], u3[...],
                     preferred_element_type=jnp.float32) + b3_ref[...]
        a = jnp.maximum(
            jnp.dot(h3, f1w_ref[...], preferred_element_type=jnp.float32)
            + f1b_ref[...], 0.0)
        a = jnp.maximum(
            jnp.dot(a, f2w_ref[...], preferred_element_type=jnp.float32)
            + f2b_ref[...], 0.0)
        out_ref[...] = (jnp.dot(a, f3w_ref[...],
                                preferred_element_type=jnp.float32)
                        + f3b_ref[...])


def _full(shape):
    return pl.BlockSpec(shape, lambda p, i: (0,) * len(shape))


def kernel(x, adj, W1, b1, W2, b2, W3, b3,
           fc1_w, fc1_b, fc2_w, fc2_b, fc3_w, fc3_b):
    f1w = jnp.zeros((128, 256), jnp.float32).at[:, :152].set(fc1_w.T)
    f1b = jnp.zeros((1, 256), jnp.float32).at[0, :152].set(fc1_b)
    f2w = jnp.zeros((256, 128), jnp.float32).at[:152, :48].set(fc2_w.T)
    f2b = jnp.zeros((1, 128), jnp.float32).at[0, :48].set(fc2_b)
    f3w = jnp.zeros((128, 128), jnp.float32).at[:48, :1].set(fc3_w.T)
    f3b = jnp.zeros((1, 128), jnp.float32).at[0, :1].set(fc3_b)

    bf = jnp.bfloat16
    stream = lambda w: pl.BlockSpec(
        (BM, w), lambda p, i: (jnp.where(p == 0, i, 0), 0))

    out = pl.pallas_call(
        _body,
        grid=(4, NB),
        in_specs=[stream(512), stream(N),
                  _full((512, 1024)), _full((1, 1024)),
                  _full((1024, 512)), _full((1, 512)),
                  _full((512, 128)), _full((1, 128)),
                  _full((128, 256)), _full((1, 256)),
                  _full((256, 128)), _full((1, 128)),
                  _full((128, 128)), _full((1, 128))],
        out_specs=pl.BlockSpec((BM, 128),
                               lambda p, i: (jnp.where(p == 3, i, 0), 0)),
        out_shape=jax.ShapeDtypeStruct((N, 128), jnp.float32),
        scratch_shapes=[pltpu.VMEM((N, N), bf),
                        pltpu.VMEM((N, 512), bf),
                        pltpu.VMEM((N, 512), bf),
                        pltpu.VMEM((N, 128), bf)],
        compiler_params=pltpu.CompilerParams(
            dimension_semantics=("arbitrary", "arbitrary"),
            vmem_limit_bytes=100 * 1024 * 1024,
        ),
    )(x, adj, W1, b1.reshape(1, -1), W2, b2.reshape(1, -1),
      W3, b3.reshape(1, -1), f1w, f1b, f2w, f2b, f3w, f3b)
    return out[:, :1]


# speedup vs baseline: 5.7346x; 1.1092x over previous
import jax
import jax.numpy as jnp
from jax.experimental import pallas as pl
from jax.experimental.pallas import tpu as pltpu

N = 4096
BM = 256
NB = N // BM


def _body(adj_ref, x_ref, w1_ref, b1_ref, w2_ref, b2_ref, w3_ref, b3_ref,
          f1w_ref, f1b_ref, f2w_ref, f2b_ref, f3w_ref, f3b_ref,
          out_ref, adj_bf, u2, u3):
    p = pl.program_id(0)
    i = pl.program_id(1)
    rows = pl.ds(i * BM, BM)
    bf = jnp.bfloat16

    @pl.when(p == 0)
    def _layer1():
        blk = adj_ref[...].astype(bf)
        adj_bf[rows, :] = blk
        a1 = jnp.dot(blk, x_ref[...], preferred_element_type=jnp.float32)
        h1 = jnp.tanh(jnp.dot(a1.astype(bf), w1_ref[...],
                              preferred_element_type=jnp.float32) + b1_ref[...])
        u2[rows, :] = jnp.dot(h1.astype(bf), w2_ref[...],
                              preferred_element_type=jnp.float32).astype(bf)

    @pl.when(p == 1)
    def _layer2():
        a2 = jnp.dot(adj_bf[rows, :], u2[...],
                     preferred_element_type=jnp.float32)
        h2 = jnp.tanh(a2 + b2_ref[...])
        u3[rows, :] = jnp.dot(h2.astype(bf), w3_ref[...],
                              preferred_element_type=jnp.float32).astype(bf)

    @pl.when(p == 2)
    def _layer3_head():
        h3 = jnp.dot(adj_bf[rows, :], u3[...],
                     preferred_element_type=jnp.float32) + b3_ref[...]
        a = jnp.maximum(
            jnp.dot(h3, f1w_ref[...], preferred_element_type=jnp.float32)
            + f1b_ref[...], 0.0)
        a = jnp.maximum(
            jnp.dot(a, f2w_ref[...], preferred_element_type=jnp.float32)
            + f2b_ref[...], 0.0)
        out_ref[...] = (jnp.dot(a, f3w_ref[...],
                                preferred_element_type=jnp.float32)
                        + f3b_ref[...])


def _full(shape):
    return pl.BlockSpec(shape, lambda p, i: (0,) * len(shape))


def kernel(x, adj, W1, b1, W2, b2, W3, b3,
           fc1_w, fc1_b, fc2_w, fc2_b, fc3_w, fc3_b):
    bf = jnp.bfloat16
    f1w = jnp.zeros((128, 256), jnp.float32).at[:, :152].set(fc1_w.T)
    f1b = jnp.zeros((1, 256), jnp.float32).at[0, :152].set(fc1_b)
    f2w = jnp.zeros((256, 128), jnp.float32).at[:152, :48].set(fc2_w.T)
    f2b = jnp.zeros((1, 128), jnp.float32).at[0, :48].set(fc2_b)
    f3w = jnp.zeros((128, 128), jnp.float32).at[:48, :1].set(fc3_w.T)
    f3b = jnp.zeros((1, 128), jnp.float32).at[0, :1].set(fc3_b)

    adj_stream = pl.BlockSpec((BM, N),
                              lambda p, i: (jnp.where(p == 0, i, 0), 0))
    out = pl.pallas_call(
        _body,
        grid=(3, NB),
        in_specs=[adj_stream, _full((N, 512)),
                  _full((512, 1024)), _full((1, 1024)),
                  _full((1024, 512)), _full((1, 512)),
                  _full((512, 128)), _full((1, 128)),
                  _full((128, 256)), _full((1, 256)),
                  _full((256, 128)), _full((1, 128)),
                  _full((128, 128)), _full((1, 128))],
        out_specs=pl.BlockSpec((BM, 128),
                               lambda p, i: (jnp.where(p == 2, i, 0), 0)),
        out_shape=jax.ShapeDtypeStruct((N, 128), jnp.float32),
        scratch_shapes=[pltpu.VMEM((N, N), bf),
                        pltpu.VMEM((N, 512), bf),
                        pltpu.VMEM((N, 128), bf)],
        compiler_params=pltpu.CompilerParams(
            dimension_semantics=("arbitrary", "arbitrary"),
            vmem_limit_bytes=100 * 1024 * 1024,
        ),
    )(adj, x.astype(bf), W1.astype(bf), b1.reshape(1, -1),
      W2.astype(bf), b2.reshape(1, -1), W3.astype(bf), b3.reshape(1, -1),
      f1w, f1b, f2w, f2b, f3w, f3b)
    return out[:, :1]


# speedup vs baseline: 6.3539x; 1.1080x over previous
import jax
import jax.numpy as jnp
from jax.experimental import pallas as pl
from jax.experimental.pallas import tpu as pltpu

N = 4096
BM = 512
NB = N // BM


def _body(adj_ref, x_ref, w1_ref, b1_ref, w2_ref, b2_ref, w3_ref, b3_ref,
          f1w_ref, f1b_ref, f2w_ref, f2b_ref, f3w_ref, f3b_ref,
          out_ref, adj_bf, u2, u3):
    p = pl.program_id(0)
    i = pl.program_id(1)
    rows = pl.ds(i * BM, BM)
    bf = jnp.bfloat16

    @pl.when(p == 0)
    def _layer1():
        blk = adj_ref[...].astype(bf)
        adj_bf[rows, :] = blk
        a1 = jnp.dot(blk, x_ref[...], preferred_element_type=jnp.float32)
        h1 = jnp.tanh(jnp.dot(a1.astype(bf), w1_ref[...],
                              preferred_element_type=jnp.float32) + b1_ref[...])
        u2[rows, :] = jnp.dot(h1.astype(bf), w2_ref[...],
                              preferred_element_type=jnp.float32).astype(bf)

    @pl.when(p == 1)
    def _layer2():
        a2 = jnp.dot(adj_bf[rows, :], u2[...],
                     preferred_element_type=jnp.float32)
        h2 = jnp.tanh(a2 + b2_ref[...])
        u3[rows, :] = jnp.dot(h2.astype(bf), w3_ref[...],
                              preferred_element_type=jnp.float32).astype(bf)

    @pl.when(p == 2)
    def _layer3_head():
        h3 = jnp.dot(adj_bf[rows, :], u3[...],
                     preferred_element_type=jnp.float32) + b3_ref[...]
        a = jnp.maximum(
            jnp.dot(h3, f1w_ref[...], preferred_element_type=jnp.float32)
            + f1b_ref[...], 0.0)
        a = jnp.maximum(
            jnp.dot(a, f2w_ref[...], preferred_element_type=jnp.float32)
            + f2b_ref[...], 0.0)
        out_ref[...] = (jnp.dot(a, f3w_ref[...],
                                preferred_element_type=jnp.float32)
                        + f3b_ref[...])


def _full(shape):
    return pl.BlockSpec(shape, lambda p, i: (0,) * len(shape))


def kernel(x, adj, W1, b1, W2, b2, W3, b3,
           fc1_w, fc1_b, fc2_w, fc2_b, fc3_w, fc3_b):
    bf = jnp.bfloat16
    f1w = jnp.zeros((128, 256), jnp.float32).at[:, :152].set(fc1_w.T)
    f1b = jnp.zeros((1, 256), jnp.float32).at[0, :152].set(fc1_b)
    f2w = jnp.zeros((256, 128), jnp.float32).at[:152, :48].set(fc2_w.T)
    f2b = jnp.zeros((1, 128), jnp.float32).at[0, :48].set(fc2_b)
    f3w = jnp.zeros((128, 128), jnp.float32).at[:48, :1].set(fc3_w.T)
    f3b = jnp.zeros((1, 128), jnp.float32).at[0, :1].set(fc3_b)

    adj_stream = pl.BlockSpec((BM, N),
                              lambda p, i: (jnp.where(p == 0, i, 0), 0))
    out = pl.pallas_call(
        _body,
        grid=(3, NB),
        in_specs=[adj_stream, _full((N, 512)),
                  _full((512, 1024)), _full((1, 1024)),
                  _full((1024, 512)), _full((1, 512)),
                  _full((512, 128)), _full((1, 128)),
                  _full((128, 256)), _full((1, 256)),
                  _full((256, 128)), _full((1, 128)),
                  _full((128, 128)), _full((1, 128))],
        out_specs=pl.BlockSpec((BM, 128),
                               lambda p, i: (jnp.where(p == 2, i, 0), 0)),
        out_shape=jax.ShapeDtypeStruct((N, 128), jnp.float32),
        scratch_shapes=[pltpu.VMEM((N, N), bf),
                        pltpu.VMEM((N, 512), bf),
                        pltpu.VMEM((N, 128), bf)],
        compiler_params=pltpu.CompilerParams(
            dimension_semantics=("arbitrary", "arbitrary"),
            vmem_limit_bytes=100 * 1024 * 1024,
        ),
    )(adj, x.astype(bf), W1.astype(bf), b1.reshape(1, -1),
      W2.astype(bf), b2.reshape(1, -1), W3.astype(bf), b3.reshape(1, -1),
      f1w, f1b, f2w, f2b, f3w, f3b)
    return out[:, :1]
